# baseline (speedup 1.0000x reference)
import functools

import jax
import jax.numpy as jnp
from jax import lax
from jax.experimental import pallas as pl
from jax.experimental.pallas import tpu as pltpu

N_NODES = 10000
D = 128
BN = 2000


def _layer_body(p_ref, ni_ref, w_ref, b_ref, o_ref, *, relu):
    h = (p_ref[0] + p_ref[1]) * ni_ref[...]
    out = lax.dot_general(
        h, w_ref[...], (((1,), (0,)), ((), ())),
        precision=lax.Precision.HIGHEST,
        preferred_element_type=jnp.float32,
    ) + b_ref[...]
    if relu:
        out = jnp.maximum(out, 0.0)
    o_ref[...] = out


def _tc_layer(p, ni, W, b, relu):
    n, dout = p.shape[1], W.shape[1]
    grid = (n // BN,)
    return pl.pallas_call(
        functools.partial(_layer_body, relu=relu),
        grid=grid,
        in_specs=[
            pl.BlockSpec((2, BN, D), lambda i: (0, i, 0)),
            pl.BlockSpec((BN, 1), lambda i: (i, 0)),
            pl.BlockSpec((D, dout), lambda i: (0, 0)),
            pl.BlockSpec((1, dout), lambda i: (0, 0)),
        ],
        out_specs=pl.BlockSpec((BN, dout), lambda i: (i, 0)),
        out_shape=jax.ShapeDtypeStruct((n, dout), jnp.float32),
    )(p, ni, W, b)


def _agg_placeholder(h, src, dst, ew2):
    m = h[src] * ew2[:, None]
    agg = jax.ops.segment_sum(m, dst, num_segments=N_NODES)
    return jnp.stack([agg, jnp.zeros_like(agg)])


def kernel(features, norm_edge_weight, edge_index, W1, b1, W2, b2, W3, b3):
    src = edge_index[0]
    dst = edge_index[1]
    ones = jnp.ones((src.shape[0],), dtype=jnp.float32)
    deg_out = jnp.clip(jax.ops.segment_sum(ones, src, num_segments=N_NODES), 1.0, None)
    deg_in = jnp.clip(jax.ops.segment_sum(ones, dst, num_segments=N_NODES), 1.0, None)
    no = deg_out ** -0.5
    ni = (deg_in ** -0.5).reshape(N_NODES, 1)
    ew2 = norm_edge_weight * no[src]

    b1r = b1.reshape(1, -1)
    b2r = b2.reshape(1, -1)
    b3r = b3.reshape(1, -1)

    p = _agg_placeholder(features, src, dst, ew2)
    h = _tc_layer(p, ni, W1, b1r, True)
    p = _agg_placeholder(h, src, dst, ew2)
    h = _tc_layer(p, ni, W2, b2r, True)
    p = _agg_placeholder(h, src, dst, ew2)
    return _tc_layer(p, ni, W3, b3r, False)


# speedup vs baseline: 1.8636x; 1.8636x over previous
import functools

import jax
import jax.numpy as jnp
from jax import lax
from jax.experimental import pallas as pl
from jax.experimental.pallas import tpu as pltpu
from jax.experimental.pallas import tpu_sc as plsc

N_NODES = 10000
D = 128
BN = 2000

NC = 2
NS = 16
NW = NC * NS
CH = 128
E_RAW = 320000
CHUNKS_PER_W = -(-E_RAW // (NW * CH))
E_PAD = NW * CH * CHUNKS_PER_W
EDGES_PER_W = CH * CHUNKS_PER_W
N_PAD = 10240
ROWS_PER_TILE = N_PAD // NS

_mesh = plsc.VectorSubcoreMesh(core_axis_name="c", subcore_axis_name="s")


def _agg_body(h_hbm, src_hbm, dst_hbm, ew_hbm, out_hbm,
              srcv, dstv, rows, zbuf, ews, acc, sem):
    cid = lax.axis_index("c")
    sid = lax.axis_index("s")
    wid = sid * NC + cid

    @pl.loop(0, CH)
    def _(i):
        for j in range(0, D, 16):
            zbuf[i, pl.ds(j, 16)] = jnp.zeros((16,), jnp.float32)

    base = sid * ROWS_PER_TILE
    for k in range(ROWS_PER_TILE // CH):
        pltpu.sync_copy(zbuf.at[pl.ds(0, CH), :],
                        acc.at[pl.ds(base + k * CH, CH), :])
    plsc.subcore_barrier()

    ebase = wid * EDGES_PER_W

    @pl.loop(0, CHUNKS_PER_W)
    def _(t):
        b0 = ebase + t * CH
        pltpu.sync_copy(src_hbm.at[pl.ds(b0, CH)], srcv)
        pltpu.sync_copy(dst_hbm.at[pl.ds(b0, CH)], dstv)
        pltpu.sync_copy(ew_hbm.at[pl.ds(b0, CH)], ews)
        pltpu.async_copy(h_hbm.at[srcv], rows, sem).wait()

        @pl.loop(0, CH, step=16)
        def _(e0):
            wv = ews[pl.ds(e0, 16)]
            for k in range(16):
                w = wv[k]
                for j in range(0, D, 16):
                    rows[e0 + k, pl.ds(j, 16)] = rows[e0 + k, pl.ds(j, 16)] * w

        pltpu.sync_copy(rows, acc.at[dstv], add=True)

    plsc.subcore_barrier()
    pltpu.sync_copy(acc.at[pl.ds(base, ROWS_PER_TILE), :],
                    out_hbm.at[cid].at[pl.ds(base, ROWS_PER_TILE), :])


_sc_agg = pl.kernel(
    _agg_body, mesh=_mesh,
    out_type=jax.ShapeDtypeStruct((NC, N_PAD, D), jnp.float32),
    scratch_types=[
        pltpu.VMEM((CH,), jnp.int32),
        pltpu.VMEM((CH,), jnp.int32),
        pltpu.VMEM((CH, D), jnp.float32),
        pltpu.VMEM((CH, D), jnp.float32),
        pltpu.VMEM((CH,), jnp.float32),
        pltpu.VMEM_SHARED((N_PAD, D), jnp.float32),
        pltpu.SemaphoreType.DMA,
    ],
)


def _layer_body(p_ref, ni_ref, w_ref, b_ref, o_ref, *, relu):
    h = (p_ref[0] + p_ref[1]) * ni_ref[...]
    out = lax.dot_general(
        h, w_ref[...], (((1,), (0,)), ((), ())),
        precision=lax.Precision.HIGHEST,
        preferred_element_type=jnp.float32,
    ) + b_ref[...]
    if relu:
        out = jnp.maximum(out, 0.0)
    o_ref[...] = out


def _tc_layer(p, ni, W, b, relu):
    n, dout = p.shape[1], W.shape[1]
    grid = (n // BN,)
    return pl.pallas_call(
        functools.partial(_layer_body, relu=relu),
        grid=grid,
        in_specs=[
            pl.BlockSpec((2, BN, D), lambda i: (0, i, 0)),
            pl.BlockSpec((BN, 1), lambda i: (i, 0)),
            pl.BlockSpec((D, dout), lambda i: (0, 0)),
            pl.BlockSpec((1, dout), lambda i: (0, 0)),
        ],
        out_specs=pl.BlockSpec((BN, dout), lambda i: (i, 0)),
        out_shape=jax.ShapeDtypeStruct((n, dout), jnp.float32),
    )(p, ni, W, b)


def kernel(features, norm_edge_weight, edge_index, W1, b1, W2, b2, W3, b3):
    src = edge_index[0]
    dst = edge_index[1]
    ones = jnp.ones((src.shape[0],), dtype=jnp.float32)
    deg_out = jnp.clip(jax.ops.segment_sum(ones, src, num_segments=N_NODES), 1.0, None)
    deg_in = jnp.clip(jax.ops.segment_sum(ones, dst, num_segments=N_NODES), 1.0, None)
    no = deg_out ** -0.5
    ni = (deg_in ** -0.5).reshape(N_NODES, 1)
    ew2 = norm_edge_weight * no[src]

    pad = E_PAD - E_RAW
    srcp = jnp.pad(src, (0, pad))
    dstp = jnp.pad(dst, (0, pad))
    ewp = jnp.pad(ew2, (0, pad))

    b1r = b1.reshape(1, -1)
    b2r = b2.reshape(1, -1)
    b3r = b3.reshape(1, -1)

    p = _sc_agg(features, srcp, dstp, ewp)[:, :N_NODES]
    h = _tc_layer(p, ni, W1, b1r, True)
    p = _sc_agg(h, srcp, dstp, ewp)[:, :N_NODES]
    h = _tc_layer(p, ni, W2, b2r, True)
    p = _sc_agg(h, srcp, dstp, ewp)[:, :N_NODES]
    return _tc_layer(p, ni, W3, b3r, False)


# speedup vs baseline: 4.6443x; 2.4921x over previous
import jax
import jax.numpy as jnp
from jax import lax
from jax.experimental import pallas as pl
from jax.experimental.pallas import tpu as pltpu
from jax.experimental.pallas import tpu_sc as plsc

N_NODES = 10000
N_PAD = 10240
D = 128
BN = 2048

NC = 2
NS = 16
NW = NC * NS
CH = 128
E_RAW = 320000
CPW = 80
E_PAD = NW * CH * CPW
EPW = CH * CPW
ROWS_PER_TILE = N_PAD // NS
PAD_NODE = 10200

_mesh = plsc.VectorSubcoreMesh(core_axis_name="c", subcore_axis_name="s")


def _deg_body(src_hbm, dst_hbm, out_hbm, ones1, z1,
              s0, d0, s1, d1, s2, d2, s3, d3,
              m0, m1, m2, m3, acc_o, acc_i, sem):
    cid = lax.axis_index("c")
    sid = lax.axis_index("s")
    wid = sid * NC + cid
    ebase = wid * EPW

    @pl.loop(0, CH, step=16)
    def _(i):
        ones1[pl.ds(i, 16)] = jnp.ones((16,), jnp.float32)

    @pl.loop(0, ROWS_PER_TILE, step=16)
    def _(i):
        z1[pl.ds(i, 16)] = jnp.zeros((16,), jnp.float32)

    base = sid * ROWS_PER_TILE
    pltpu.sync_copy(z1, acc_o.at[pl.ds(base, ROWS_PER_TILE)])
    pltpu.sync_copy(z1, acc_i.at[pl.ds(base, ROWS_PER_TILE)])
    plsc.subcore_barrier()

    srcs = (s0, s1, s2, s3)
    dsts = (d0, d1, d2, d3)
    sems = (m0, m1, m2, m3)

    def prefetch(t, j):
        pltpu.async_copy(src_hbm.at[pl.ds(ebase + t * CH, CH)], srcs[j], sems[j])
        pltpu.async_copy(dst_hbm.at[pl.ds(ebase + t * CH, CH)], dsts[j], sems[j])

    def wait_idx(j):
        pltpu.make_async_copy(src_hbm.at[pl.ds(0, CH)], srcs[j], sems[j]).wait()
        pltpu.make_async_copy(dst_hbm.at[pl.ds(0, CH)], dsts[j], sems[j]).wait()

    for j in range(4):
        prefetch(j, j)

    @pl.loop(0, CPW, step=4)
    def _(t):
        for j in range(4):
            wait_idx(j)
            pltpu.sync_copy(ones1, acc_o.at[srcs[j]], add=True)
            pltpu.sync_copy(ones1, acc_i.at[dsts[j]], add=True)

            @pl.when(t + j + 4 < CPW)
            def _():
                prefetch(t + j + 4, j)

    plsc.subcore_barrier()
    pltpu.sync_copy(acc_o.at[pl.ds(base, ROWS_PER_TILE)],
                    out_hbm.at[cid, 0].at[pl.ds(base, ROWS_PER_TILE)])
    pltpu.sync_copy(acc_i.at[pl.ds(base, ROWS_PER_TILE)],
                    out_hbm.at[cid, 1].at[pl.ds(base, ROWS_PER_TILE)])


_sc_deg = pl.kernel(
    _deg_body, mesh=_mesh,
    out_type=jax.ShapeDtypeStruct((NC, 2, N_PAD), jnp.float32),
    scratch_types=(
        [pltpu.VMEM((CH,), jnp.float32),
         pltpu.VMEM((ROWS_PER_TILE,), jnp.float32)]
        + [pltpu.VMEM((CH,), jnp.int32)] * 8
        + [pltpu.SemaphoreType.DMA] * 4
        + [pltpu.VMEM_SHARED((N_PAD,), jnp.float32),
           pltpu.VMEM_SHARED((N_PAD,), jnp.float32),
           pltpu.SemaphoreType.DMA]
    ),
)


def _agg_body(h_hbm, src_hbm, dst_hbm, ew_hbm, out_hbm,
              s0, d0, e0, s1, d1, e1, s2, d2, e2, s3, d3, e3,
              m0, m1, m2, m3,
              rows_a, rows_b, acc, ga, gb):
    cid = lax.axis_index("c")
    sid = lax.axis_index("s")
    wid = sid * NC + cid
    ebase = wid * EPW

    @pl.loop(0, CH)
    def _(i):
        for j in range(0, D, 16):
            rows_a[i, pl.ds(j, 16)] = jnp.zeros((16,), jnp.float32)

    base = sid * ROWS_PER_TILE
    for k in range(ROWS_PER_TILE // CH):
        pltpu.sync_copy(rows_a, acc.at[pl.ds(base + k * CH, CH), :])
    plsc.subcore_barrier()

    srcs = (s0, s1, s2, s3)
    dsts = (d0, d1, d2, d3)
    ews = (e0, e1, e2, e3)
    sems = (m0, m1, m2, m3)
    rows = (rows_a, rows_b)
    gsems = (ga, gb)

    def prefetch(t, j):
        off = ebase + t * CH
        pltpu.async_copy(src_hbm.at[pl.ds(off, CH)], srcs[j], sems[j])
        pltpu.async_copy(dst_hbm.at[pl.ds(off, CH)], dsts[j], sems[j])
        pltpu.async_copy(ew_hbm.at[pl.ds(off, CH)], ews[j], sems[j])

    def wait_idx(j):
        pltpu.make_async_copy(src_hbm.at[pl.ds(0, CH)], srcs[j], sems[j]).wait()
        pltpu.make_async_copy(dst_hbm.at[pl.ds(0, CH)], dsts[j], sems[j]).wait()
        pltpu.make_async_copy(ew_hbm.at[pl.ds(0, CH)], ews[j], sems[j]).wait()

    def gather(j, r):
        pltpu.async_copy(h_hbm.at[srcs[j]], rows[r], gsems[r])

    def wait_gather(r):
        pltpu.make_async_copy(h_hbm.at[pl.ds(0, CH), :], rows[r], gsems[r]).wait()

    def scale(j, r):
        @pl.loop(0, CH, step=16)
        def _(c0):
            wv = ews[j][pl.ds(c0, 16)]
            for k in range(16):
                w = wv[k]
                for q in range(0, D, 16):
                    rows[r][c0 + k, pl.ds(q, 16)] = (
                        rows[r][c0 + k, pl.ds(q, 16)] * w)

    for j in range(4):
        prefetch(j, j)
    wait_idx(0)
    gather(0, 0)
    wait_idx(1)
    gather(1, 1)

    @pl.loop(0, CPW, step=4)
    def _(t):
        for j in range(4):
            r = j % 2
            wait_gather(r)
            scale(j, r)
            pltpu.sync_copy(rows[r], acc.at[dsts[j]], add=True)

            @pl.when(t + j + 4 < CPW)
            def _():
                prefetch(t + j + 4, j)

            @pl.when(t + j + 2 < CPW)
            def _():
                j2 = (j + 2) % 4
                wait_idx(j2)
                gather(j2, r)

    plsc.subcore_barrier()
    pltpu.sync_copy(acc.at[pl.ds(base, ROWS_PER_TILE), :],
                    out_hbm.at[cid].at[pl.ds(base, ROWS_PER_TILE), :])


_sc_agg = pl.kernel(
    _agg_body, mesh=_mesh,
    out_type=jax.ShapeDtypeStruct((NC, N_PAD, D), jnp.float32),
    scratch_types=(
        [pltpu.VMEM((CH,), jnp.int32), pltpu.VMEM((CH,), jnp.int32),
         pltpu.VMEM((CH,), jnp.float32)] * 4
        + [pltpu.SemaphoreType.DMA] * 4
        + [pltpu.VMEM((CH, D), jnp.float32), pltpu.VMEM((CH, D), jnp.float32),
           pltpu.VMEM_SHARED((N_PAD, D), jnp.float32),
           pltpu.SemaphoreType.DMA, pltpu.SemaphoreType.DMA]
    ),
)


def _norm_body(d_ref, f_ref, n_ref, f2_ref):
    s = d_ref[0] + d_ref[1]
    nv = lax.rsqrt(jnp.maximum(s, 1.0))
    n_ref[...] = nv
    f2_ref[...] = f_ref[...] * nv[0][:, None]


def _tc_norms(degp, featp):
    grid = (N_PAD // BN,)
    return pl.pallas_call(
        _norm_body,
        grid=grid,
        in_specs=[
            pl.BlockSpec((2, 2, BN), lambda i: (0, 0, i)),
            pl.BlockSpec((BN, D), lambda i: (i, 0)),
        ],
        out_specs=[
            pl.BlockSpec((2, BN), lambda i: (0, i)),
            pl.BlockSpec((BN, D), lambda i: (i, 0)),
        ],
        out_shape=[
            jax.ShapeDtypeStruct((2, N_PAD), jnp.float32),
            jax.ShapeDtypeStruct((N_PAD, D), jnp.float32),
        ],
    )(degp, featp)


def _layer_body(fl_ref, p_ref, ni_ref, no_ref, w_ref, b_ref, o_ref):
    h = (p_ref[0] + p_ref[1]) * ni_ref[...]
    out = lax.dot_general(
        h, w_ref[...], (((1,), (0,)), ((), ())),
        precision=lax.Precision.HIGHEST,
        preferred_element_type=jnp.float32,
    ) + b_ref[...]
    o_ref[...] = jnp.where(fl_ref[0, 0] > 0,
                           no_ref[...] * jnp.maximum(out, 0.0), out)


def _tc_layer(p, ni, no, W, b, fl):
    grid = (N_PAD // BN,)
    return pl.pallas_call(
        _layer_body,
        grid=grid,
        in_specs=[
            pl.BlockSpec((1, 1), lambda i: (0, 0)),
            pl.BlockSpec((2, BN, D), lambda i: (0, i, 0)),
            pl.BlockSpec((BN, 1), lambda i: (i, 0)),
            pl.BlockSpec((BN, 1), lambda i: (i, 0)),
            pl.BlockSpec((D, D), lambda i: (0, 0)),
            pl.BlockSpec((1, D), lambda i: (0, 0)),
        ],
        out_specs=pl.BlockSpec((BN, D), lambda i: (i, 0)),
        out_shape=jax.ShapeDtypeStruct((N_PAD, D), jnp.float32),
    )(fl, p, ni, no, W, b)


def kernel(features, norm_edge_weight, edge_index, W1, b1, W2, b2, W3, b3):
    src = edge_index[0]
    dst = edge_index[1]
    pad = E_PAD - E_RAW
    srcp = jnp.pad(src, (0, pad), constant_values=PAD_NODE)
    dstp = jnp.pad(dst, (0, pad), constant_values=PAD_NODE)
    ewp = jnp.pad(norm_edge_weight, (0, pad))
    featp = jnp.pad(features, ((0, N_PAD - N_NODES), (0, 0)))

    degp = _sc_deg(srcp, dstp)
    norms, feat0 = _tc_norms(degp, featp)
    no = norms[0].reshape(N_PAD, 1)
    ni = norms[1].reshape(N_PAD, 1)

    n_classes = W3.shape[1]
    Ws = jnp.stack([W1, W2, jnp.pad(W3, ((0, 0), (0, D - n_classes)))])
    bs = jnp.stack([b1, b2, jnp.pad(b3, (0, D - n_classes))]).reshape(3, 1, D)
    fls = jnp.array([1.0, 1.0, 0.0], jnp.float32).reshape(3, 1, 1)

    def body(h, xs):
        W, b, fl = xs
        p = _sc_agg(h, srcp, dstp, ewp)
        return _tc_layer(p, ni, no, W, b, fl), None

    h_final, _ = lax.scan(body, feat0, (Ws, bs, fls))
    return h_final[:N_NODES, :n_classes]


# speedup vs baseline: 5.0434x; 1.0859x over previous
import jax
import jax.numpy as jnp
from jax import lax
from jax.experimental import pallas as pl
from jax.experimental.pallas import tpu as pltpu
from jax.experimental.pallas import tpu_sc as plsc

N_NODES = 10000
N_PAD = 10240
D = 128
BN = 2048

NC = 2
NS = 16
NW = NC * NS
CH = 128
E_RAW = 320000
CPW = 80
CPW0 = 124
CPW1 = 36
E_PAD = NW * CH * CPW
EPW = CH * CPW
ROWS_PER_TILE = N_PAD // NS
PAD_NODE = 10200

_mesh = plsc.VectorSubcoreMesh(core_axis_name="c", subcore_axis_name="s")


def _deg_body(src_hbm, dst_hbm, out_hbm, ones1, z1,
              s0, d0, s1, d1, s2, d2, s3, d3,
              m0, m1, m2, m3, acc_o, acc_i, sem):
    cid = lax.axis_index("c")
    sid = lax.axis_index("s")
    wid = sid * NC + cid
    ebase = wid * EPW

    @pl.loop(0, CH, step=16)
    def _(i):
        ones1[pl.ds(i, 16)] = jnp.ones((16,), jnp.float32)

    @pl.loop(0, ROWS_PER_TILE, step=16)
    def _(i):
        z1[pl.ds(i, 16)] = jnp.zeros((16,), jnp.float32)

    base = sid * ROWS_PER_TILE
    pltpu.sync_copy(z1, acc_o.at[pl.ds(base, ROWS_PER_TILE)])
    pltpu.sync_copy(z1, acc_i.at[pl.ds(base, ROWS_PER_TILE)])
    plsc.subcore_barrier()

    srcs = (s0, s1, s2, s3)
    dsts = (d0, d1, d2, d3)
    sems = (m0, m1, m2, m3)

    def prefetch(t, j):
        pltpu.async_copy(src_hbm.at[pl.ds(ebase + t * CH, CH)], srcs[j], sems[j])
        pltpu.async_copy(dst_hbm.at[pl.ds(ebase + t * CH, CH)], dsts[j], sems[j])

    def wait_idx(j):
        pltpu.make_async_copy(src_hbm.at[pl.ds(0, CH)], srcs[j], sems[j]).wait()
        pltpu.make_async_copy(dst_hbm.at[pl.ds(0, CH)], dsts[j], sems[j]).wait()

    for j in range(4):
        prefetch(j, j)

    @pl.loop(0, CPW, step=4)
    def _(t):
        for j in range(4):
            wait_idx(j)
            pltpu.sync_copy(ones1, acc_o.at[srcs[j]], add=True)
            pltpu.sync_copy(ones1, acc_i.at[dsts[j]], add=True)

            @pl.when(t + j + 4 < CPW)
            def _():
                prefetch(t + j + 4, j)

    plsc.subcore_barrier()
    pltpu.sync_copy(acc_o.at[pl.ds(base, ROWS_PER_TILE)],
                    out_hbm.at[cid, 0].at[pl.ds(base, ROWS_PER_TILE)])
    pltpu.sync_copy(acc_i.at[pl.ds(base, ROWS_PER_TILE)],
                    out_hbm.at[cid, 1].at[pl.ds(base, ROWS_PER_TILE)])


_sc_deg = pl.kernel(
    _deg_body, mesh=_mesh,
    out_type=jax.ShapeDtypeStruct((NC, 2, N_PAD), jnp.float32),
    scratch_types=(
        [pltpu.VMEM((CH,), jnp.float32),
         pltpu.VMEM((ROWS_PER_TILE,), jnp.float32)]
        + [pltpu.VMEM((CH,), jnp.int32)] * 8
        + [pltpu.SemaphoreType.DMA] * 4
        + [pltpu.VMEM_SHARED((N_PAD,), jnp.float32),
           pltpu.VMEM_SHARED((N_PAD,), jnp.float32),
           pltpu.SemaphoreType.DMA]
    ),
)


def _agg_body(h_hbm, src_hbm, dst_hbm, ew_hbm, out_hbm,
              s0, d0, e0, s1, d1, e1, s2, d2, e2, s3, d3, e3,
              m0, m1, m2, m3,
              rows_a, rows_b, acc, ga, gb):
    cid = lax.axis_index("c")
    sid = lax.axis_index("s")
    cpw = jnp.where(cid == 0, CPW0, CPW1)
    ebase = jnp.where(cid == 0, sid * CPW0, NS * CPW0 + sid * CPW1) * CH

    @pl.loop(0, CH)
    def _(i):
        for j in range(0, D, 16):
            rows_a[i, pl.ds(j, 16)] = jnp.zeros((16,), jnp.float32)

    base = sid * ROWS_PER_TILE
    for k in range(ROWS_PER_TILE // CH):
        pltpu.sync_copy(rows_a, acc.at[pl.ds(base + k * CH, CH), :])
    plsc.subcore_barrier()

    srcs = (s0, s1, s2, s3)
    dsts = (d0, d1, d2, d3)
    ews = (e0, e1, e2, e3)
    sems = (m0, m1, m2, m3)
    rows = (rows_a, rows_b)
    gsems = (ga, gb)

    def prefetch(t, j):
        off = ebase + t * CH
        pltpu.async_copy(src_hbm.at[pl.ds(off, CH)], srcs[j], sems[j])
        pltpu.async_copy(dst_hbm.at[pl.ds(off, CH)], dsts[j], sems[j])
        pltpu.async_copy(ew_hbm.at[pl.ds(off, CH)], ews[j], sems[j])

    def wait_idx(j):
        pltpu.make_async_copy(src_hbm.at[pl.ds(0, CH)], srcs[j], sems[j]).wait()
        pltpu.make_async_copy(dst_hbm.at[pl.ds(0, CH)], dsts[j], sems[j]).wait()
        pltpu.make_async_copy(ew_hbm.at[pl.ds(0, CH)], ews[j], sems[j]).wait()

    def gather(j, r):
        pltpu.async_copy(h_hbm.at[srcs[j]], rows[r], gsems[r])

    def wait_gather(r):
        pltpu.make_async_copy(h_hbm.at[pl.ds(0, CH), :], rows[r], gsems[r]).wait()

    def scale(j, r):
        @pl.loop(0, CH, step=16)
        def _(c0):
            wv = ews[j][pl.ds(c0, 16)]
            for k in range(16):
                w = wv[k]
                for q in range(0, D, 16):
                    rows[r][c0 + k, pl.ds(q, 16)] = (
                        rows[r][c0 + k, pl.ds(q, 16)] * w)

    for j in range(4):
        prefetch(j, j)
    wait_idx(0)
    gather(0, 0)
    wait_idx(1)
    gather(1, 1)

    @pl.loop(0, cpw, step=4)
    def _(t):
        for j in range(4):
            r = j % 2
            wait_gather(r)
            scale(j, r)
            pltpu.sync_copy(rows[r], acc.at[dsts[j]], add=True)

            @pl.when(t + j + 4 < cpw)
            def _():
                prefetch(t + j + 4, j)

            @pl.when(t + j + 2 < cpw)
            def _():
                j2 = (j + 2) % 4
                wait_idx(j2)
                gather(j2, r)

    plsc.subcore_barrier()
    pltpu.sync_copy(acc.at[pl.ds(base, ROWS_PER_TILE), :],
                    out_hbm.at[cid].at[pl.ds(base, ROWS_PER_TILE), :])


_sc_agg = pl.kernel(
    _agg_body, mesh=_mesh,
    out_type=jax.ShapeDtypeStruct((NC, N_PAD, D), jnp.float32),
    scratch_types=(
        [pltpu.VMEM((CH,), jnp.int32), pltpu.VMEM((CH,), jnp.int32),
         pltpu.VMEM((CH,), jnp.float32)] * 4
        + [pltpu.SemaphoreType.DMA] * 4
        + [pltpu.VMEM((CH, D), jnp.float32), pltpu.VMEM((CH, D), jnp.float32),
           pltpu.VMEM_SHARED((N_PAD, D), jnp.float32),
           pltpu.SemaphoreType.DMA, pltpu.SemaphoreType.DMA]
    ),
)


def _norm_body(d_ref, f_ref, n_ref, f2_ref):
    s = d_ref[0] + d_ref[1]
    nv = lax.rsqrt(jnp.maximum(s, 1.0))
    n_ref[...] = nv
    f2_ref[...] = f_ref[...] * nv[0][:, None]


def _tc_norms(degp, featp):
    grid = (N_PAD // BN,)
    return pl.pallas_call(
        _norm_body,
        grid=grid,
        in_specs=[
            pl.BlockSpec((2, 2, BN), lambda i: (0, 0, i)),
            pl.BlockSpec((BN, D), lambda i: (i, 0)),
        ],
        out_specs=[
            pl.BlockSpec((2, BN), lambda i: (0, i)),
            pl.BlockSpec((BN, D), lambda i: (i, 0)),
        ],
        out_shape=[
            jax.ShapeDtypeStruct((2, N_PAD), jnp.float32),
            jax.ShapeDtypeStruct((N_PAD, D), jnp.float32),
        ],
    )(degp, featp)


def _layer_body(fl_ref, p_ref, ni_ref, no_ref, w_ref, b_ref, o_ref):
    h = (p_ref[0] + p_ref[1]) * ni_ref[...]
    out = lax.dot_general(
        h, w_ref[...], (((1,), (0,)), ((), ())),
        precision=lax.Precision.HIGHEST,
        preferred_element_type=jnp.float32,
    ) + b_ref[...]
    o_ref[...] = jnp.where(fl_ref[0, 0] > 0,
                           no_ref[...] * jnp.maximum(out, 0.0), out)


def _tc_layer(p, ni, no, W, b, fl):
    grid = (N_PAD // BN,)
    return pl.pallas_call(
        _layer_body,
        grid=grid,
        in_specs=[
            pl.BlockSpec((1, 1), lambda i: (0, 0)),
            pl.BlockSpec((2, BN, D), lambda i: (0, i, 0)),
            pl.BlockSpec((BN, 1), lambda i: (i, 0)),
            pl.BlockSpec((BN, 1), lambda i: (i, 0)),
            pl.BlockSpec((D, D), lambda i: (0, 0)),
            pl.BlockSpec((1, D), lambda i: (0, 0)),
        ],
        out_specs=pl.BlockSpec((BN, D), lambda i: (i, 0)),
        out_shape=jax.ShapeDtypeStruct((N_PAD, D), jnp.float32),
    )(fl, p, ni, no, W, b)


def kernel(features, norm_edge_weight, edge_index, W1, b1, W2, b2, W3, b3):
    src = edge_index[0]
    dst = edge_index[1]
    pad = E_PAD - E_RAW
    srcp = jnp.pad(src, (0, pad), constant_values=PAD_NODE)
    dstp = jnp.pad(dst, (0, pad), constant_values=PAD_NODE)
    ewp = jnp.pad(norm_edge_weight, (0, pad))
    featp = jnp.pad(features, ((0, N_PAD - N_NODES), (0, 0)))

    degp = _sc_deg(srcp, dstp)
    norms, feat0 = _tc_norms(degp, featp)
    no = norms[0].reshape(N_PAD, 1)
    ni = norms[1].reshape(N_PAD, 1)

    n_classes = W3.shape[1]
    Ws = jnp.stack([W1, W2, jnp.pad(W3, ((0, 0), (0, D - n_classes)))])
    bs = jnp.stack([b1, b2, jnp.pad(b3, (0, D - n_classes))]).reshape(3, 1, D)
    fls = jnp.array([1.0, 1.0, 0.0], jnp.float32).reshape(3, 1, 1)

    def body(h, xs):
        W, b, fl = xs
        p = _sc_agg(h, srcp, dstp, ewp)
        return _tc_layer(p, ni, no, W, b, fl), None

    h_final, _ = lax.scan(body, feat0, (Ws, bs, fls))
    return h_final[:N_NODES, :n_classes]
